# baseline (speedup 1.0000x reference)
import jax
import jax.numpy as jnp
from jax import lax
from jax.experimental import pallas as pl
from jax.experimental.pallas import tpu as pltpu
from jax.experimental.pallas import tpu_sc as plsc

NC = 2
NS = 16
NW = NC * NS
L = 16

EPS = 1e-12


def _rsqrt(x):
    i = lax.bitcast_convert_type(x, jnp.int32)
    i = jnp.int32(0x5F3759DF) - (i >> 1)
    y = lax.bitcast_convert_type(i, jnp.float32)
    for _ in range(3):
        y = y * (1.5 - 0.5 * x * y * y)
    return y


def _make_kernel(B, S, E, rows_per_w):
    EB = E // L
    SP = ((S + L - 1) // L) * L
    NFULL = S // L
    NREM = S % L
    N = rows_per_w
    NB3 = (N + 2) // 3

    def body(ids_hbm, tt_hbm, word_hbm, pos_hbm, ttemb_hbm, gamma_hbm,
             beta_hbm, out_hbm,
             comb_v, ttemb_v, gam_v, bet_v,
             ids0, ids1, ids2, ttid0, ttid1, ttid2,
             rows0, rows1, rows2,
             gsem0, gsem1, gsem2, osem0, osem1, osem2):
        wid = lax.axis_index("s") * NC + lax.axis_index("c")
        base_row = wid * N

        slots = [
            (ids0, ttid0, rows0, gsem0, osem0),
            (ids1, ttid1, rows1, gsem1, osem1),
            (ids2, ttid2, rows2, gsem2, osem2),
        ]

        def ids_load(c, sl):
            ids_v, ttid_v = sl[0], sl[1]
            row = base_row + c
            pltpu.sync_copy(ids_hbm.at[pl.ds(row * S, S)], ids_v)
            pltpu.sync_copy(tt_hbm.at[pl.ds(row * S, S)],
                            ttid_v.at[pl.ds(0, S)])

        def gather_copies(sl):
            ids_v, rows_v, gsem = sl[0], sl[2], sl[3]
            c0 = pltpu.make_async_copy(
                word_hbm.at[ids_v.at[pl.ds(0, 128)]],
                rows_v.at[pl.ds(0, 128)], gsem)
            c1 = pltpu.make_async_copy(
                word_hbm.at[ids_v.at[pl.ds(128, S - 128)]],
                rows_v.at[pl.ds(128, S - 128)], gsem)
            return c0, c1

        def gather_start(sl):
            for cp in gather_copies(sl):
                cp.start()

        def gather_wait(sl):
            for cp in gather_copies(sl):
                cp.wait()

        def out_copy(c, sl):
            rows_v, osem = sl[2], sl[4]
            row = base_row + c
            return pltpu.make_async_copy(
                rows_v, out_hbm.at[pl.ds(row * S, S)], osem)

        pltpu.sync_copy(pos_hbm.at[pl.ds(0, S)], comb_v.at[pl.ds(0, S)])
        pltpu.sync_copy(pos_hbm.at[pl.ds(0, S)], comb_v.at[pl.ds(S, S)])
        pltpu.sync_copy(ttemb_hbm, ttemb_v)
        pltpu.sync_copy(gamma_hbm, gam_v)
        pltpu.sync_copy(beta_hbm, bet_v)

        tte0 = [ttemb_v[0, pl.ds(e * L, L)] for e in range(EB)]
        tte1 = [ttemb_v[1, pl.ds(e * L, L)] for e in range(EB)]

        def build_body(s, cc):
            for e in range(EB):
                comb_v[s, pl.ds(e * L, L)] = (
                    comb_v[s, pl.ds(e * L, L)] + tte0[e])
                comb_v[S + s, pl.ds(e * L, L)] = (
                    comb_v[S + s, pl.ds(e * L, L)] + tte1[e])
            return cc

        lax.fori_loop(0, S, build_body, 0)

        gam = [gam_v[pl.ds(e * L, L)] for e in range(EB)]
        bet = [bet_v[pl.ds(e * L, L)] for e in range(EB)]

        def compute(sl):
            ttid_v, rows_v = sl[1], sl[2]

            def token(j, tt):
                cb = tt * S + j
                v = []
                for e in range(EB):
                    x = rows_v[j, pl.ds(e * L, L)]
                    p = comb_v[cb, pl.ds(e * L, L)]
                    v.append(x + p)
                sv = v[0] + v[1]
                for e in range(2, EB):
                    sv = sv + v[e]
                qv = v[0] * v[0]
                for e in range(1, EB):
                    qv = qv + v[e] * v[e]
                s1 = jnp.broadcast_to(jnp.sum(sv), (L,))
                s2 = jnp.broadcast_to(jnp.sum(qv), (L,))
                mean = s1 * (1.0 / E)
                var = s2 * (1.0 / E) - mean * mean
                r = _rsqrt(var + EPS)
                for e in range(EB):
                    rg = r * gam[e]
                    cst = bet[e] - mean * rg
                    rows_v[j, pl.ds(e * L, L)] = v[e] * rg + cst

            def blk_body(b, cc):
                tv = ttid_v[pl.ds(b * L, L)]
                for k in range(L):
                    token(b * L + k, tv[k])
                return cc

            lax.fori_loop(0, NFULL, blk_body, 0)
            if NREM:
                tv = ttid_v[pl.ds(NFULL * L, L)]
                for k in range(NREM):
                    token(NFULL * L + k, tv[k])

        ids_load(0, slots[0])
        ids_load(1, slots[1])
        ids_load(2, slots[2])
        gather_start(slots[0])
        gather_start(slots[1])

        def loop_body(p, carry):
            cb = p * 3
            for k in range(3):
                c = cb + k
                sl = slots[k]
                sl2 = slots[(k + 2) % 3]

                @pl.when(c < N)
                def _():
                    gather_wait(sl)
                    compute(sl)
                    out_copy(c, sl).start()

                @pl.when(c + 3 < N)
                def _():
                    ids_load(c + 3, sl)

                @pl.when((c >= 1) & (c < N))
                def _():
                    out_copy(c - 1, sl2).wait()

                @pl.when(c + 2 < N)
                def _():
                    gather_start(sl2)
            return carry

        lax.fori_loop(0, NB3, loop_body, 0)
        out_copy(N - 1, slots[(N - 1) % 3]).wait()

    mesh = plsc.VectorSubcoreMesh(core_axis_name="c", subcore_axis_name="s")
    return pl.kernel(
        body,
        out_type=jax.ShapeDtypeStruct((B * S, E), jnp.float32),
        mesh=mesh,
        compiler_params=pltpu.CompilerParams(needs_layout_passes=False),
        scratch_types=[
            pltpu.VMEM((2 * S, E), jnp.float32),
            pltpu.VMEM((2, E), jnp.float32),
            pltpu.VMEM((E,), jnp.float32),
            pltpu.VMEM((E,), jnp.float32),
            pltpu.VMEM((S,), jnp.int32),
            pltpu.VMEM((S,), jnp.int32),
            pltpu.VMEM((S,), jnp.int32),
            pltpu.VMEM((SP,), jnp.int32),
            pltpu.VMEM((SP,), jnp.int32),
            pltpu.VMEM((SP,), jnp.int32),
            pltpu.VMEM((S, E), jnp.float32),
            pltpu.VMEM((S, E), jnp.float32),
            pltpu.VMEM((S, E), jnp.float32),
            pltpu.SemaphoreType.DMA,
            pltpu.SemaphoreType.DMA,
            pltpu.SemaphoreType.DMA,
            pltpu.SemaphoreType.DMA,
            pltpu.SemaphoreType.DMA,
            pltpu.SemaphoreType.DMA,
        ],
    )


@jax.jit
def kernel(input_ids, token_type_ids, word_embeddings, position_embeddings,
           token_type_embeddings, gamma, beta):
    B, S = input_ids.shape
    E = word_embeddings.shape[1]
    rows_per_w = B // NW
    k = _make_kernel(B, S, E, rows_per_w)
    out = k(input_ids.astype(jnp.int32).reshape(-1),
            token_type_ids.astype(jnp.int32).reshape(-1),
            word_embeddings, position_embeddings, token_type_embeddings,
            gamma, beta)
    return out.reshape(B, S, E)


# speedup vs baseline: 1.9164x; 1.9164x over previous
import jax
import jax.numpy as jnp
from jax import lax
from jax.experimental import pallas as pl
from jax.experimental.pallas import tpu as pltpu
from jax.experimental.pallas import tpu_sc as plsc

NC = 2
NS = 16
NW = NC * NS
L = 16

EPS = 1e-12


def _rsqrt(x):
    i = lax.bitcast_convert_type(x, jnp.int32)
    i = jnp.int32(0x5F3759DF) - (i >> 1)
    y = lax.bitcast_convert_type(i, jnp.float32)
    for _ in range(3):
        y = y * (1.5 - 0.5 * x * y * y)
    return y


def _make_kernel(B, S, E, rows_per_w):
    EB = E // L
    SP = ((S + L - 1) // L) * L
    NFULL = S // L
    NREM = S % L
    N = rows_per_w
    NB3 = (N + 2) // 3

    def body(ids_hbm, tt_hbm, word_hbm, pos_hbm, ttemb_hbm, gamma_hbm,
             beta_hbm, out_hbm,
             pos_v, ttemb_v, gam_v, bet_v,
             ids0, ids1, ids2, ttid0, ttid1, ttid2,
             rows0, rows1, rows2,
             gsem0, gsem1, gsem2, osem0, osem1, osem2):
        wid = lax.axis_index("s") * NC + lax.axis_index("c")
        base_row = wid * N

        slots = [
            (ids0, ttid0, rows0, gsem0, osem0),
            (ids1, ttid1, rows1, gsem1, osem1),
            (ids2, ttid2, rows2, gsem2, osem2),
        ]

        def ids_load(c, sl):
            ids_v, ttid_v = sl[0], sl[1]
            row = base_row + c
            pltpu.sync_copy(ids_hbm.at[pl.ds(row * S, S)], ids_v)
            pltpu.sync_copy(tt_hbm.at[pl.ds(row * S, S)],
                            ttid_v.at[pl.ds(0, S)])

        def gather_copies(sl):
            ids_v, rows_v, gsem = sl[0], sl[2], sl[3]
            c0 = pltpu.make_async_copy(
                word_hbm.at[ids_v.at[pl.ds(0, 128)]],
                rows_v.at[pl.ds(0, 128)], gsem)
            c1 = pltpu.make_async_copy(
                word_hbm.at[ids_v.at[pl.ds(128, S - 128)]],
                rows_v.at[pl.ds(128, S - 128)], gsem)
            return c0, c1

        def gather_start(sl):
            for cp in gather_copies(sl):
                cp.start()

        def gather_wait(sl):
            for cp in gather_copies(sl):
                cp.wait()

        def out_copy(c, sl):
            rows_v, osem = sl[2], sl[4]
            row = base_row + c
            return pltpu.make_async_copy(
                rows_v, out_hbm.at[pl.ds(row * S, S)], osem)

        pltpu.sync_copy(pos_hbm.at[pl.ds(0, S)], pos_v)
        pltpu.sync_copy(ttemb_hbm, ttemb_v)
        pltpu.sync_copy(gamma_hbm, gam_v)
        pltpu.sync_copy(beta_hbm, bet_v)

        tte0 = [ttemb_v[0, pl.ds(e * L, L)] for e in range(EB)]
        tte1 = [ttemb_v[1, pl.ds(e * L, L)] for e in range(EB)]
        dlt = [tte1[e] - tte0[e] for e in range(EB)]

        def build_body(s, cc):
            for e in range(EB):
                pos_v[s, pl.ds(e * L, L)] = (
                    pos_v[s, pl.ds(e * L, L)] + tte0[e])
            return cc

        lax.fori_loop(0, S, build_body, 0)

        gam = [gam_v[pl.ds(e * L, L)] for e in range(EB)]
        bet = [bet_v[pl.ds(e * L, L)] for e in range(EB)]
        lane = jnp.arange(L, dtype=jnp.int32)

        def compute(sl):
            ttid_v, rows_v = sl[1], sl[2]

            def block(b, nt):
                tv = ttid_v[pl.ds(b * L, L)]
                fv = tv.astype(jnp.float32)
                ps1 = jnp.zeros((L,), jnp.float32)
                ps2 = jnp.zeros((L,), jnp.float32)
                for k in range(nt):
                    j = b * L + k
                    fk = jnp.broadcast_to(fv[k], (L,))
                    v = []
                    for e in range(EB):
                        x = rows_v[j, pl.ds(e * L, L)]
                        p = pos_v[j, pl.ds(e * L, L)]
                        v.append((x + p) + fk * dlt[e])
                    sv = v[0] + v[1]
                    for e in range(2, EB):
                        sv = sv + v[e]
                    qv = v[0] * v[0]
                    for e in range(1, EB):
                        qv = qv + v[e] * v[e]
                    for e in range(EB):
                        rows_v[j, pl.ds(e * L, L)] = v[e]
                    s1 = jnp.broadcast_to(jnp.sum(sv), (L,))
                    s2 = jnp.broadcast_to(jnp.sum(qv), (L,))
                    ps1 = jnp.where(lane == k, s1, ps1)
                    ps2 = jnp.where(lane == k, s2, ps2)
                mean16 = ps1 * (1.0 / E)
                var16 = ps2 * (1.0 / E) - mean16 * mean16
                r16 = _rsqrt(var16 + EPS)
                for k in range(nt):
                    j = b * L + k
                    rb = jnp.broadcast_to(r16[k], (L,))
                    mb = jnp.broadcast_to(mean16[k], (L,))
                    for e in range(EB):
                        x = rows_v[j, pl.ds(e * L, L)]
                        rg = rb * gam[e]
                        cst = bet[e] - mb * rg
                        rows_v[j, pl.ds(e * L, L)] = x * rg + cst

            def blk_body(b, cc):
                block(b, L)
                return cc

            lax.fori_loop(0, NFULL, blk_body, 0)
            if NREM:
                block(NFULL, NREM)

        ids_load(0, slots[0])
        ids_load(1, slots[1])
        ids_load(2, slots[2])
        gather_start(slots[0])
        gather_start(slots[1])

        def loop_body(p, carry):
            cb = p * 3
            for k in range(3):
                c = cb + k
                sl = slots[k]
                sl2 = slots[(k + 2) % 3]

                @pl.when(c < N)
                def _():
                    gather_wait(sl)
                    compute(sl)
                    out_copy(c, sl).start()

                @pl.when(c + 3 < N)
                def _():
                    ids_load(c + 3, sl)

                @pl.when((c >= 1) & (c < N))
                def _():
                    out_copy(c - 1, sl2).wait()

                @pl.when(c + 2 < N)
                def _():
                    gather_start(sl2)
            return carry

        lax.fori_loop(0, NB3, loop_body, 0)
        out_copy(N - 1, slots[(N - 1) % 3]).wait()

    mesh = plsc.VectorSubcoreMesh(core_axis_name="c", subcore_axis_name="s")
    return pl.kernel(
        body,
        out_type=jax.ShapeDtypeStruct((B * S, E), jnp.float32),
        mesh=mesh,
        compiler_params=pltpu.CompilerParams(needs_layout_passes=False),
        scratch_types=[
            pltpu.VMEM((S, E), jnp.float32),
            pltpu.VMEM((2, E), jnp.float32),
            pltpu.VMEM((E,), jnp.float32),
            pltpu.VMEM((E,), jnp.float32),
            pltpu.VMEM((S,), jnp.int32),
            pltpu.VMEM((S,), jnp.int32),
            pltpu.VMEM((S,), jnp.int32),
            pltpu.VMEM((SP,), jnp.int32),
            pltpu.VMEM((SP,), jnp.int32),
            pltpu.VMEM((SP,), jnp.int32),
            pltpu.VMEM((S, E), jnp.float32),
            pltpu.VMEM((S, E), jnp.float32),
            pltpu.VMEM((S, E), jnp.float32),
            pltpu.SemaphoreType.DMA,
            pltpu.SemaphoreType.DMA,
            pltpu.SemaphoreType.DMA,
            pltpu.SemaphoreType.DMA,
            pltpu.SemaphoreType.DMA,
            pltpu.SemaphoreType.DMA,
        ],
    )


@jax.jit
def kernel(input_ids, token_type_ids, word_embeddings, position_embeddings,
           token_type_embeddings, gamma, beta):
    B, S = input_ids.shape
    E = word_embeddings.shape[1]
    rows_per_w = B // NW
    k = _make_kernel(B, S, E, rows_per_w)
    out = k(input_ids.astype(jnp.int32).reshape(-1),
            token_type_ids.astype(jnp.int32).reshape(-1),
            word_embeddings, position_embeddings, token_type_embeddings,
            gamma, beta)
    return out.reshape(B, S, E)


# speedup vs baseline: 1.9228x; 1.0034x over previous
import jax
import jax.numpy as jnp
from jax import lax
from jax.experimental import pallas as pl
from jax.experimental.pallas import tpu as pltpu
from jax.experimental.pallas import tpu_sc as plsc

NC = 2
NS = 16
NW = NC * NS
L = 16

EPS = 1e-12


def _rsqrt(x):
    i = lax.bitcast_convert_type(x, jnp.int32)
    i = jnp.int32(0x5F3759DF) - (i >> 1)
    y = lax.bitcast_convert_type(i, jnp.float32)
    for _ in range(3):
        y = y * (1.5 - 0.5 * x * y * y)
    return y


def _make_kernel(B, S, E, rows_per_w):
    EB = E // L
    SP = ((S + L - 1) // L) * L
    NFULL = S // L
    NREM = S % L
    NBLK = SP // L
    N = rows_per_w
    NB4 = (N + 3) // 4

    def body(ids_hbm, tt_hbm, word_hbm, comb_hbm, gamma_hbm, beta_hbm,
             out_hbm,
             gam_v, bet_v, ttid_v,
             ids0, ids1, ids2, ids3, idx0, idx1, idx2, idx3,
             rows0, rows1, rows2, rows3,
             gsem0, gsem1, gsem2, gsem3,
             asem0, asem1, asem2, asem3,
             osem0, osem1, osem2, osem3):
        wid = lax.axis_index("s") * NC + lax.axis_index("c")
        base_row = wid * N

        slots = [
            (ids0, idx0, rows0, gsem0, asem0, osem0),
            (ids1, idx1, rows1, gsem1, asem1, osem1),
            (ids2, idx2, rows2, gsem2, asem2, osem2),
            (ids3, idx3, rows3, gsem3, asem3, osem3),
        ]

        lane = jnp.arange(L, dtype=jnp.int32)

        def ids_load(c, sl):
            ids_v, idx_v = sl[0], sl[1]
            row = base_row + c
            pltpu.sync_copy(ids_hbm.at[pl.ds(row * S, S)], ids_v)
            pltpu.sync_copy(tt_hbm.at[pl.ds(row * S, S)],
                            ttid_v.at[pl.ds(0, S)])

            def idx_body(b, cc):
                tv = ttid_v[pl.ds(b * L, L)]
                idx_v[pl.ds(b * L, L)] = tv * S + (lane + b * L)
                return cc

            lax.fori_loop(0, NBLK, idx_body, 0)

        def indirect_copies(table, sl, idx_pos, sem_pos):
            iv, rows_v, sem = sl[idx_pos], sl[2], sl[sem_pos]
            c0 = pltpu.make_async_copy(
                table.at[iv.at[pl.ds(0, 128)]],
                rows_v.at[pl.ds(0, 128)], sem)
            c1 = pltpu.make_async_copy(
                table.at[iv.at[pl.ds(128, S - 128)]],
                rows_v.at[pl.ds(128, S - 128)], sem)
            return c0, c1

        def word_start(sl):
            for cp in indirect_copies(word_hbm, sl, 0, 3):
                cp.start()

        def word_wait(sl):
            for cp in indirect_copies(word_hbm, sl, 0, 3):
                cp.wait()

        def comb_start(sl):
            for cp in indirect_copies(comb_hbm, sl, 1, 4):
                cp.start(add=True)

        def comb_wait(sl):
            for cp in indirect_copies(comb_hbm, sl, 1, 4):
                cp.wait()

        def out_copy(c, sl):
            rows_v, osem = sl[2], sl[5]
            row = base_row + c
            return pltpu.make_async_copy(
                rows_v, out_hbm.at[pl.ds(row * S, S)], osem)

        pltpu.sync_copy(gamma_hbm, gam_v)
        pltpu.sync_copy(beta_hbm, bet_v)
        gam = [gam_v[pl.ds(e * L, L)] for e in range(EB)]
        bet = [bet_v[pl.ds(e * L, L)] for e in range(EB)]

        def compute(sl):
            rows_v = sl[2]

            def block(b, nt):
                ps1 = jnp.zeros((L,), jnp.float32)
                ps2 = jnp.zeros((L,), jnp.float32)
                for k in range(nt):
                    j = b * L + k
                    v = [rows_v[j, pl.ds(e * L, L)] for e in range(EB)]
                    sv = v[0] + v[1]
                    for e in range(2, EB):
                        sv = sv + v[e]
                    qv = v[0] * v[0]
                    for e in range(1, EB):
                        qv = qv + v[e] * v[e]
                    s1 = jnp.broadcast_to(jnp.sum(sv), (L,))
                    s2 = jnp.broadcast_to(jnp.sum(qv), (L,))
                    ps1 = jnp.where(lane == k, s1, ps1)
                    ps2 = jnp.where(lane == k, s2, ps2)
                mean16 = ps1 * (1.0 / E)
                var16 = ps2 * (1.0 / E) - mean16 * mean16
                r16 = _rsqrt(var16 + EPS)
                for k in range(nt):
                    j = b * L + k
                    rb = jnp.broadcast_to(r16[k], (L,))
                    mb = jnp.broadcast_to(mean16[k], (L,))
                    for e in range(EB):
                        x = rows_v[j, pl.ds(e * L, L)]
                        rg = rb * gam[e]
                        cst = bet[e] - mb * rg
                        rows_v[j, pl.ds(e * L, L)] = x * rg + cst

            def blk_body(b, cc):
                block(b, L)
                return cc

            lax.fori_loop(0, NFULL, blk_body, 0)
            if NREM:
                block(NFULL, NREM)

        ids_load(0, slots[0])
        ids_load(1, slots[1])
        ids_load(2, slots[2])
        word_start(slots[0])
        word_start(slots[1])
        word_wait(slots[0])
        comb_start(slots[0])

        def loop_body(p, carry):
            cb = p * 4
            for k in range(4):
                c = cb + k
                sl = slots[k]
                sl1 = slots[(k + 1) % 4]
                sl2 = slots[(k + 2) % 4]
                sl3 = slots[(k + 3) % 4]

                @pl.when(c + 2 < N)
                def _():
                    @pl.when(c >= 2)
                    def _():
                        out_copy(c - 2, sl2).wait()
                    word_start(sl2)

                @pl.when(c + 1 < N)
                def _():
                    word_wait(sl1)
                    comb_start(sl1)

                @pl.when(c < N)
                def _():
                    comb_wait(sl)
                    compute(sl)
                    out_copy(c, sl).start()

                @pl.when(c + 3 < N)
                def _():
                    ids_load(c + 3, sl3)
            return carry

        lax.fori_loop(0, NB4, loop_body, 0)
        out_copy(N - 2, slots[(N - 2) % 4]).wait()
        out_copy(N - 1, slots[(N - 1) % 4]).wait()

    mesh = plsc.VectorSubcoreMesh(core_axis_name="c", subcore_axis_name="s")
    return pl.kernel(
        body,
        out_type=jax.ShapeDtypeStruct((B * S, E), jnp.float32),
        mesh=mesh,
        compiler_params=pltpu.CompilerParams(needs_layout_passes=False),
        scratch_types=[
            pltpu.VMEM((E,), jnp.float32),
            pltpu.VMEM((E,), jnp.float32),
            pltpu.VMEM((SP,), jnp.int32),
            pltpu.VMEM((S,), jnp.int32),
            pltpu.VMEM((S,), jnp.int32),
            pltpu.VMEM((S,), jnp.int32),
            pltpu.VMEM((S,), jnp.int32),
            pltpu.VMEM((SP,), jnp.int32),
            pltpu.VMEM((SP,), jnp.int32),
            pltpu.VMEM((SP,), jnp.int32),
            pltpu.VMEM((SP,), jnp.int32),
            pltpu.VMEM((S, E), jnp.float32),
            pltpu.VMEM((S, E), jnp.float32),
            pltpu.VMEM((S, E), jnp.float32),
            pltpu.VMEM((S, E), jnp.float32),
            pltpu.SemaphoreType.DMA,
            pltpu.SemaphoreType.DMA,
            pltpu.SemaphoreType.DMA,
            pltpu.SemaphoreType.DMA,
            pltpu.SemaphoreType.DMA,
            pltpu.SemaphoreType.DMA,
            pltpu.SemaphoreType.DMA,
            pltpu.SemaphoreType.DMA,
            pltpu.SemaphoreType.DMA,
            pltpu.SemaphoreType.DMA,
            pltpu.SemaphoreType.DMA,
            pltpu.SemaphoreType.DMA,
        ],
    )


@jax.jit
def kernel(input_ids, token_type_ids, word_embeddings, position_embeddings,
           token_type_embeddings, gamma, beta):
    B, S = input_ids.shape
    E = word_embeddings.shape[1]
    rows_per_w = B // NW
    comb = (token_type_embeddings[:, None, :]
            + position_embeddings[None, :S, :]).reshape(2 * S, E)
    k = _make_kernel(B, S, E, rows_per_w)
    out = k(input_ids.astype(jnp.int32).reshape(-1),
            token_type_ids.astype(jnp.int32).reshape(-1),
            word_embeddings, comb, gamma, beta)
    return out.reshape(B, S, E)
